# baseline (speedup 1.0000x reference)
import functools

import jax
import jax.numpy as jnp
from jax import lax
from jax.experimental import pallas as pl
from jax.experimental.pallas import tpu as pltpu
from jax.experimental.pallas import tpu_sc as plsc

_B, _T = 16384, 50
_D = 64
_N_EMB = 1000000
_NC, _NS = 2, 16
_NW = _NC * _NS
_BPW = _B // _NW
_BH = 256
_NU = _T * (_BPW // _BH)
_L = 16

_mesh = plsc.VectorSubcoreMesh(core_axis_name="c", subcore_axis_name="s")

_NBLK = (_N_EMB + 127) // 128
_NFULL = _N_EMB // 128


@functools.partial(
    pl.kernel,
    mesh=_mesh,
    out_type=jax.ShapeDtypeStruct((_N_EMB, 128), jnp.float32),
    scratch_types=[
        pltpu.VMEM((2, _D, 128), jnp.float32),
        pltpu.VMEM((2, 128, 128), jnp.float32),
        pltpu.SemaphoreType.DMA,
        pltpu.SemaphoreType.DMA,
        pltpu.SemaphoreType.DMA,
        pltpu.SemaphoreType.DMA,
    ],
    compiler_params=pltpu.CompilerParams(
        use_tc_tiling_on_sc=True, needs_layout_passes=False
    ),
)
def _relayout_kernel(embt_hbm, tail_hbm, out_hbm, vin, vtr, g0, g1, s0, s1):
    gsem = (g0, g1)
    ssem = (s0, s1)
    wid = lax.axis_index("s") * _NC + lax.axis_index("c")
    nblk = (_NFULL - wid + _NW - 1) // _NW

    iota = lax.iota(jnp.int32, _L)
    rots = [jnp.bitwise_and(iota + k, _L - 1) for k in range(_L)]

    def fire(c, s):
        pltpu.async_copy(
            embt_hbm.at[:, pl.ds(c * 128, 128)], vin.at[s], gsem[s]
        )

    def wait_gather(s):
        pltpu.make_async_copy(
            embt_hbm.at[:, pl.ds(0, 128)], vin.at[s], gsem[s]
        ).wait()

    def wait_store(s):
        pltpu.make_async_copy(
            out_hbm.at[pl.ds(0, 128)], vtr.at[s], ssem[s]
        ).wait()

    def transpose_block(s, nb):
        def body(bg, _):
            bvec = bg * _L + iota
            for d0 in range(0, _D, _L):
                for k in range(_L):
                    dvec = rots[k] + d0
                    v = plsc.load_gather(vin.at[s], [dvec, bvec])
                    plsc.store_scatter(vtr.at[s], [bvec, dvec], v)
            return 0

        lax.fori_loop(0, nb, body, 0, unroll=2)

    @pl.when(nblk > 0)
    def _():
        fire(wid, 0)

    @pl.when(nblk > 1)
    def _():
        fire(wid + _NW, 1)

    def step(j, _):
        s = lax.rem(j, 2)
        c = wid + j * _NW

        def do(s):
            wait_gather(s)

            @pl.when(j >= 2)
            def _():
                wait_store(s)

            transpose_block(s, 8)
            pltpu.async_copy(
                vtr.at[s], out_hbm.at[pl.ds(c * 128, 128)], ssem[s]
            )
            nxt = j + 2

            @pl.when(nxt < nblk)
            def _():
                fire(wid + nxt * _NW, s)

        @pl.when(s == 0)
        def _():
            do(0)

        @pl.when(s == 1)
        def _():
            do(1)

        return 0

    lax.fori_loop(0, nblk, step, 0, unroll=False)

    @pl.when(nblk > 0)
    def _():
        wait_store(0)

    @pl.when(nblk > 1)
    def _():
        wait_store(1)

    @pl.when(wid == 4)
    def _():
        pltpu.sync_copy(tail_hbm, vtr.at[0, pl.ds(0, 64)])
        pltpu.sync_copy(
            vtr.at[0, pl.ds(0, 64)],
            out_hbm.at[pl.ds(_NFULL * 128, 64)],
        )


@functools.partial(
    pl.kernel,
    mesh=_mesh,
    out_type=jax.ShapeDtypeStruct((_T, _D, _B), jnp.float32),
    scratch_types=[
        pltpu.VMEM((_NU * 2, 128), jnp.int32),
        pltpu.VMEM((2, _BH, 128), jnp.float32),
        pltpu.VMEM((2, _D, _BH), jnp.float32),
        pltpu.SemaphoreType.DMA,
        pltpu.SemaphoreType.DMA,
        pltpu.SemaphoreType.DMA,
        pltpu.SemaphoreType.DMA,
    ],
    compiler_params=pltpu.CompilerParams(
        use_tc_tiling_on_sc=True, needs_layout_passes=False
    ),
)
def _gather_kernel(
    emb_hbm, idx_hbm, out_hbm, idx_v, in_v, tr_v, g0, g1, s0, s1
):
    gsem = (g0, g1)
    ssem = (s0, s1)
    wid = lax.axis_index("s") * _NC + lax.axis_index("c")
    b0 = wid * _BPW

    pltpu.sync_copy(idx_hbm.at[wid], idx_v)

    def fire_gather(u, s):
        for q in range(2):
            pltpu.async_copy(
                emb_hbm.at[idx_v.at[u * 2 + q]],
                in_v.at[s, pl.ds(q * 128, 128)],
                gsem[s],
            )

    def wait_gather(s):
        pltpu.make_async_copy(
            emb_hbm.at[pl.ds(0, _BH)], in_v.at[s], gsem[s]
        ).wait()

    def wait_store(s):
        pltpu.make_async_copy(
            out_hbm.at[0, :, pl.ds(0, _BH)], tr_v.at[s], ssem[s]
        ).wait()

    fire_gather(0, 0)
    fire_gather(1, 1)

    iota = lax.iota(jnp.int32, _L)
    rots = [jnp.bitwise_and(iota + k, _L - 1) for k in range(_L)]

    def transpose_unit(s):
        def body(bg, _):
            bvec = bg * _L + iota
            for d0 in range(0, _D, _L):
                for k in range(_L):
                    dvec = rots[k] + d0
                    v = plsc.load_gather(in_v.at[s], [bvec, dvec])
                    plsc.store_scatter(tr_v.at[s], [dvec, bvec], v)
            return 0

        lax.fori_loop(0, _BH // _L, body, 0, unroll=2)

    def step(g, h):
        u = 2 * g + h
        wait_gather(h)

        @pl.when(u >= 2)
        def _():
            wait_store(h)

        transpose_unit(h)
        pltpu.async_copy(
            tr_v.at[h],
            out_hbm.at[g, :, pl.ds(b0 + h * _BH, _BH)],
            ssem[h],
        )
        nxt = u + 2

        @pl.when(nxt < _NU)
        def _():
            fire_gather(nxt, h)

    def group(g, _):
        step(g, 0)
        step(g, 1)
        return 0

    lax.fori_loop(0, _NU // 2, group, 0, unroll=False)
    wait_store(0)
    wait_store(1)


def kernel(token_ids, embedding):
    tail = jnp.pad(embedding[_NFULL * 128 :], ((0, 0), (0, 64)))
    emb2 = _relayout_kernel(embedding.T, tail)
    idx = (
        token_ids.astype(jnp.int32)
        .reshape(_NW, _BPW, _T)
        .transpose(0, 2, 1)
        .reshape(_NW, _NU * 2, 128)
    )
    out_t = _gather_kernel(emb2, idx)
    return out_t.transpose(2, 0, 1)


# speedup vs baseline: 1.0276x; 1.0276x over previous
import functools

import jax
import jax.numpy as jnp
from jax import lax
from jax.experimental import pallas as pl
from jax.experimental.pallas import tpu as pltpu
from jax.experimental.pallas import tpu_sc as plsc

_B, _T = 16384, 50
_D = 64
_N_EMB = 1000000
_NC, _NS = 2, 16
_NW = _NC * _NS
_BPW = _B // _NW
_BH = 256
_NU = _T * (_BPW // _BH)
_L = 16

_mesh = plsc.VectorSubcoreMesh(core_axis_name="c", subcore_axis_name="s")

_NBLK = (_N_EMB + 127) // 128
_NFULL = _N_EMB // 128


@functools.partial(
    pl.kernel,
    mesh=_mesh,
    out_type=jax.ShapeDtypeStruct((_N_EMB, 128), jnp.float32),
    scratch_types=[
        pltpu.VMEM((2, _D, 128), jnp.float32),
        pltpu.VMEM((2, 128, 128), jnp.float32),
        pltpu.SemaphoreType.DMA,
        pltpu.SemaphoreType.DMA,
        pltpu.SemaphoreType.DMA,
        pltpu.SemaphoreType.DMA,
    ],
    compiler_params=pltpu.CompilerParams(
        use_tc_tiling_on_sc=True, needs_layout_passes=False
    ),
)
def _relayout_kernel(embt_hbm, tail_hbm, out_hbm, vin, vtr, g0, g1, s0, s1):
    gsem = (g0, g1)
    ssem = (s0, s1)
    wid = lax.axis_index("s") * _NC + lax.axis_index("c")
    nblk = (_NFULL - wid + _NW - 1) // _NW

    iota = lax.iota(jnp.int32, _L)
    rots = [jnp.bitwise_and(iota + k, _L - 1) for k in range(_L)]

    def fire(c, s):
        pltpu.async_copy(
            embt_hbm.at[:, pl.ds(c * 128, 128)], vin.at[s], gsem[s]
        )

    def wait_gather(s):
        pltpu.make_async_copy(
            embt_hbm.at[:, pl.ds(0, 128)], vin.at[s], gsem[s]
        ).wait()

    def wait_store(s):
        pltpu.make_async_copy(
            out_hbm.at[pl.ds(0, 128)], vtr.at[s], ssem[s]
        ).wait()

    def transpose_block(s, nb):
        @plsc.parallel_loop(0, nb, unroll=2)
        def body(bg):
            bvec = bg * _L + iota
            for d0 in range(0, _D, _L):
                for k in range(_L):
                    dvec = rots[k] + d0
                    v = plsc.load_gather(vin.at[s], [dvec, bvec])
                    plsc.store_scatter(vtr.at[s], [bvec, dvec], v)

    @pl.when(nblk > 0)
    def _():
        fire(wid, 0)

    @pl.when(nblk > 1)
    def _():
        fire(wid + _NW, 1)

    def step(j, _):
        s = lax.rem(j, 2)
        c = wid + j * _NW

        def do(s):
            wait_gather(s)

            @pl.when(j >= 2)
            def _():
                wait_store(s)

            transpose_block(s, 8)
            pltpu.async_copy(
                vtr.at[s], out_hbm.at[pl.ds(c * 128, 128)], ssem[s]
            )
            nxt = j + 2

            @pl.when(nxt < nblk)
            def _():
                fire(wid + nxt * _NW, s)

        @pl.when(s == 0)
        def _():
            do(0)

        @pl.when(s == 1)
        def _():
            do(1)

        return 0

    lax.fori_loop(0, nblk, step, 0, unroll=False)

    @pl.when(nblk > 0)
    def _():
        wait_store(0)

    @pl.when(nblk > 1)
    def _():
        wait_store(1)

    @pl.when(wid == 4)
    def _():
        pltpu.sync_copy(tail_hbm, vtr.at[0, pl.ds(0, 64)])
        pltpu.sync_copy(
            vtr.at[0, pl.ds(0, 64)],
            out_hbm.at[pl.ds(_NFULL * 128, 64)],
        )


@functools.partial(
    pl.kernel,
    mesh=_mesh,
    out_type=jax.ShapeDtypeStruct((_T, _D, _B), jnp.float32),
    scratch_types=[
        pltpu.VMEM((_NU * 2, 128), jnp.int32),
        pltpu.VMEM((2, _BH, 128), jnp.float32),
        pltpu.VMEM((2, _D, _BH), jnp.float32),
        pltpu.SemaphoreType.DMA,
        pltpu.SemaphoreType.DMA,
        pltpu.SemaphoreType.DMA,
        pltpu.SemaphoreType.DMA,
    ],
    compiler_params=pltpu.CompilerParams(
        use_tc_tiling_on_sc=True, needs_layout_passes=False
    ),
)
def _gather_kernel(
    emb_hbm, idx_hbm, out_hbm, idx_v, in_v, tr_v, g0, g1, s0, s1
):
    gsem = (g0, g1)
    ssem = (s0, s1)
    wid = lax.axis_index("s") * _NC + lax.axis_index("c")
    b0 = wid * _BPW

    pltpu.sync_copy(idx_hbm.at[wid], idx_v)

    def fire_gather(u, s):
        for q in range(2):
            pltpu.async_copy(
                emb_hbm.at[idx_v.at[u * 2 + q]],
                in_v.at[s, pl.ds(q * 128, 128)],
                gsem[s],
            )

    def wait_gather(s):
        pltpu.make_async_copy(
            emb_hbm.at[pl.ds(0, _BH)], in_v.at[s], gsem[s]
        ).wait()

    def wait_store(s):
        pltpu.make_async_copy(
            out_hbm.at[0, :, pl.ds(0, _BH)], tr_v.at[s], ssem[s]
        ).wait()

    fire_gather(0, 0)
    fire_gather(1, 1)

    iota = lax.iota(jnp.int32, _L)
    rots = [jnp.bitwise_and(iota + k, _L - 1) for k in range(_L)]

    def transpose_unit(s):
        @plsc.parallel_loop(0, _BH // _L, unroll=2)
        def body(bg):
            bvec = bg * _L + iota
            for d0 in range(0, _D, _L):
                for k in range(_L):
                    dvec = rots[k] + d0
                    v = plsc.load_gather(in_v.at[s], [bvec, dvec])
                    plsc.store_scatter(tr_v.at[s], [dvec, bvec], v)

    def step(g, h):
        u = 2 * g + h
        wait_gather(h)

        @pl.when(u >= 2)
        def _():
            wait_store(h)

        transpose_unit(h)
        pltpu.async_copy(
            tr_v.at[h],
            out_hbm.at[g, :, pl.ds(b0 + h * _BH, _BH)],
            ssem[h],
        )
        nxt = u + 2

        @pl.when(nxt < _NU)
        def _():
            fire_gather(nxt, h)

    def group(g, _):
        step(g, 0)
        step(g, 1)
        return 0

    lax.fori_loop(0, _NU // 2, group, 0, unroll=False)
    wait_store(0)
    wait_store(1)


def kernel(token_ids, embedding):
    tail = jnp.pad(embedding[_NFULL * 128 :], ((0, 0), (0, 64)))
    emb2 = _relayout_kernel(embedding.T, tail)
    idx = (
        token_ids.astype(jnp.int32)
        .reshape(_NW, _BPW, _T)
        .transpose(0, 2, 1)
        .reshape(_NW, _NU * 2, 128)
    )
    out_t = _gather_kernel(emb2, idx)
    return out_t.transpose(2, 0, 1)


# speedup vs baseline: 1.2703x; 1.2362x over previous
import functools

import jax
import jax.numpy as jnp
from jax import lax
from jax.experimental import pallas as pl
from jax.experimental.pallas import tpu as pltpu
from jax.experimental.pallas import tpu_sc as plsc

_B, _T = 16384, 50
_D = 64
_N_EMB = 1000000
_NC, _NS = 2, 16
_NW = _NC * _NS
_BPW = _B // _NW
_BH = 256
_NU = _T * (_BPW // _BH)
_L = 16

_mesh = plsc.VectorSubcoreMesh(core_axis_name="c", subcore_axis_name="s")

_NBLK = (_N_EMB + 127) // 128
_NFULL = _N_EMB // 128


@functools.partial(
    pl.kernel,
    mesh=_mesh,
    out_type=jax.ShapeDtypeStruct((_N_EMB, 128), jnp.float32),
    scratch_types=[
        pltpu.VMEM((2, _D, 128), jnp.float32),
        pltpu.VMEM((2, 128, 128), jnp.float32),
        pltpu.SemaphoreType.DMA,
        pltpu.SemaphoreType.DMA,
        pltpu.SemaphoreType.DMA,
        pltpu.SemaphoreType.DMA,
    ],
    compiler_params=pltpu.CompilerParams(
        use_tc_tiling_on_sc=True, needs_layout_passes=False
    ),
)
def _relayout_kernel(embt_hbm, tail_hbm, out_hbm, vin, vtr, g0, g1, s0, s1):
    gsem = (g0, g1)
    ssem = (s0, s1)
    wid = lax.axis_index("s") * _NC + lax.axis_index("c")
    nblk = (_NFULL - wid + _NW - 1) // _NW

    iota = lax.iota(jnp.int32, _L)
    rots = [jnp.bitwise_and(iota + k, _L - 1) for k in range(_L)]

    def fire(c, s):
        pltpu.async_copy(
            embt_hbm.at[:, pl.ds(c * 128, 128)], vin.at[s], gsem[s]
        )

    def wait_gather(s):
        pltpu.make_async_copy(
            embt_hbm.at[:, pl.ds(0, 128)], vin.at[s], gsem[s]
        ).wait()

    def wait_store(s):
        pltpu.make_async_copy(
            out_hbm.at[pl.ds(0, 128)], vtr.at[s], ssem[s]
        ).wait()

    def transpose_block(s, nb):
        @plsc.parallel_loop(0, nb, unroll=2)
        def body(bg):
            bvec = bg * _L + iota
            for d0 in range(0, _D, _L):
                for k in range(_L):
                    dvec = rots[k] + d0
                    v = plsc.load_gather(vin.at[s], [dvec, bvec])
                    plsc.store_scatter(vtr.at[s], [bvec, dvec], v)

    @pl.when(nblk > 0)
    def _():
        fire(wid, 0)

    @pl.when(nblk > 1)
    def _():
        fire(wid + _NW, 1)

    def step(j, _):
        s = lax.rem(j, 2)
        c = wid + j * _NW

        def do(s):
            wait_gather(s)

            @pl.when(j >= 2)
            def _():
                wait_store(s)

            transpose_block(s, 8)
            pltpu.async_copy(
                vtr.at[s], out_hbm.at[pl.ds(c * 128, 128)], ssem[s]
            )
            nxt = j + 2

            @pl.when(nxt < nblk)
            def _():
                fire(wid + nxt * _NW, s)

        @pl.when(s == 0)
        def _():
            do(0)

        @pl.when(s == 1)
        def _():
            do(1)

        return 0

    lax.fori_loop(0, nblk, step, 0, unroll=False)

    @pl.when(nblk > 0)
    def _():
        wait_store(0)

    @pl.when(nblk > 1)
    def _():
        wait_store(1)

    @pl.when(wid == 4)
    def _():
        pltpu.sync_copy(tail_hbm, vtr.at[0, pl.ds(0, 64)])
        pltpu.sync_copy(
            vtr.at[0, pl.ds(0, 64)],
            out_hbm.at[pl.ds(_NFULL * 128, 64)],
        )


@functools.partial(
    pl.kernel,
    mesh=_mesh,
    out_type=jax.ShapeDtypeStruct((_T, _D, _B), jnp.float32),
    scratch_types=[
        pltpu.VMEM((_NU * 2, 128), jnp.int32),
        pltpu.VMEM((2, _BH, 128), jnp.float32),
        pltpu.VMEM((2, _D, _BH), jnp.float32),
        pltpu.SemaphoreType.DMA,
        pltpu.SemaphoreType.DMA,
        pltpu.SemaphoreType.DMA,
        pltpu.SemaphoreType.DMA,
    ],
    compiler_params=pltpu.CompilerParams(
        use_tc_tiling_on_sc=True, needs_layout_passes=False
    ),
)
def _gather_kernel(
    emb_hbm, idx_hbm, out_hbm, idx_v, in_v, tr_v, g0, g1, s0, s1
):
    gsem = (g0, g1)
    ssem = (s0, s1)
    wid = lax.axis_index("s") * _NC + lax.axis_index("c")
    b0 = wid * _BPW

    pltpu.sync_copy(idx_hbm.at[wid], idx_v)

    def fire_gather(u, s):
        for q in range(2):
            pltpu.async_copy(
                emb_hbm.at[idx_v.at[u * 2 + q]],
                in_v.at[s, pl.ds(q * 128, 128)],
                gsem[s],
            )

    def wait_gather(s):
        pltpu.make_async_copy(
            emb_hbm.at[pl.ds(0, _BH)], in_v.at[s], gsem[s]
        ).wait()

    def wait_store(s):
        pltpu.make_async_copy(
            out_hbm.at[0, :, pl.ds(0, _BH)], tr_v.at[s], ssem[s]
        ).wait()

    fire_gather(0, 0)
    fire_gather(1, 1)

    iota = lax.iota(jnp.int32, _L)
    rots = [jnp.bitwise_and(iota + k, _L - 1) for k in range(_L)]

    def transpose_unit(s):
        @plsc.parallel_loop(0, _BH // _L, unroll=2)
        def body(bg):
            bvec = bg * _L + iota
            for d0 in range(0, _D, _L):
                for k in range(_L):
                    dvec = rots[k] + d0
                    v = plsc.load_gather(in_v.at[s], [bvec, dvec])
                    plsc.store_scatter(tr_v.at[s], [dvec, bvec], v)

    def step(g, h):
        u = 2 * g + h
        wait_gather(h)

        @pl.when(u >= 2)
        def _():
            wait_store(h)

        transpose_unit(h)
        pltpu.async_copy(
            tr_v.at[h],
            out_hbm.at[g, :, pl.ds(b0 + h * _BH, _BH)],
            ssem[h],
        )
        nxt = u + 2

        @pl.when(nxt < _NU)
        def _():
            fire_gather(nxt, h)

    def group(g, _):
        step(g, 0)
        step(g, 1)
        return 0

    lax.fori_loop(0, _NU // 2, group, 0, unroll=False)
    wait_store(0)
    wait_store(1)


def kernel(token_ids, embedding):
    emb2 = jnp.pad(embedding, ((0, 0), (0, 64)))
    idx = (
        token_ids.astype(jnp.int32)
        .reshape(_NW, _BPW, _T)
        .transpose(0, 2, 1)
        .reshape(_NW, _NU * 2, 128)
    )
    out_t = _gather_kernel(emb2, idx)
    return out_t.transpose(2, 0, 1)
